# Initial kernel scaffold; baseline (speedup 1.0000x reference)
import jax
import jax.numpy as jnp
from jax.experimental import pallas as pl


def kernel(X, mask, path_lengths, path_length_embedding):
    raise NotImplementedError("write your pallas kernel here")


# speedup vs baseline: 126.6974x; 126.6974x over previous
import functools

import jax
import jax.numpy as jnp
from jax import lax
from jax.experimental import pallas as pl
from jax.experimental.pallas import tpu as pltpu
from jax.experimental.pallas import tpu_sc as plsc

B = 8
N = 511
N1 = 512
H = 8
V = 512
L = 16
NC, NS = 2, 16
NW = NC * NS
ROWS_PER_W = (B * N1) // NW
TILES_PER_B = N1 // ROWS_PER_W
G = 32
NGROUPS = ROWS_PER_W // G
GROUP_WORDS = G * N1
STAGE_LEN = ROWS_PER_W * N + 16


def _tile_body(pl_hbm, emb_hbm, out_hbm, idx_buf, out_buf, emb_v):
    wid = lax.axis_index("s") * NC + lax.axis_index("c")
    b = wid // TILES_PER_B
    i_start = (wid % TILES_PER_B) * ROWS_PER_W
    start_p = jnp.maximum(i_start - 1, 0)
    offset = (b * N + start_p) * N
    start8 = (offset // 8) * 8
    slop = offset - start8

    pltpu.sync_copy(emb_hbm, emb_v)
    pltpu.sync_copy(pl_hbm.at[pl.ds(start8, STAGE_LEN)], idx_buf)

    iota = lax.iota(jnp.int32, L)
    ones = jnp.full((L,), 1, jnp.int32)
    emb1 = plsc.load_gather(emb_v, [ones])

    for g in range(NGROUPS):
        @pl.loop(0, G)
        def _row(lr):
            i = i_start + g * G + lr

            obase = lr * N1

            @pl.when(i == 0)
            def _():
                for j in range(N1 // L):
                    out_buf[pl.ds(obase + j * L, L)] = emb1

            @pl.when(i > 0)
            def _():
                base = (i - 1 - start_p) * N + slop
                idx0 = jnp.maximum(iota + (base - 1), 0)
                v0 = plsc.load_gather(idx_buf, [idx0])
                v0 = jnp.where(iota == 0, 1, v0)
                out_buf[pl.ds(obase, L)] = plsc.load_gather(emb_v, [v0])
                for j in range(1, N1 // L):
                    idxs = iota + (base + j * L - 1)
                    vals = plsc.load_gather(idx_buf, [idxs])
                    out_buf[pl.ds(obase + j * L, L)] = plsc.load_gather(emb_v, [vals])

        row0 = i_start + g * G
        for h in range(H):
            dst_off = ((b * H + h) * N1 + row0) * N1
            pltpu.sync_copy(out_buf, out_hbm.at[pl.ds(dst_off, GROUP_WORDS)])


def kernel(X, mask, path_lengths, path_length_embedding):
    del X, mask
    pl_flat = path_lengths.reshape(-1)
    emb_flat = path_length_embedding.reshape(-1)

    mesh = plsc.VectorSubcoreMesh(
        core_axis_name="c", subcore_axis_name="s", num_cores=NC, num_subcores=NS
    )
    run = functools.partial(
        pl.kernel,
        out_type=jax.ShapeDtypeStruct((B * H * N1 * N1,), jnp.float32),
        mesh=mesh,
        scratch_types=[
            pltpu.VMEM((STAGE_LEN,), jnp.int32),
            pltpu.VMEM((GROUP_WORDS,), jnp.float32),
            pltpu.VMEM((V,), jnp.float32),
        ],
        compiler_params=pltpu.CompilerParams(needs_layout_passes=False),
    )(_tile_body)
    out = run(pl_flat, emb_flat)
    return out.reshape(B * H, N1, N1)


# speedup vs baseline: 127.4178x; 1.0057x over previous
import functools

import jax
import jax.numpy as jnp
from jax import lax
from jax.experimental import pallas as pl
from jax.experimental.pallas import tpu as pltpu
from jax.experimental.pallas import tpu_sc as plsc

B = 8
N = 511
N1 = 512
H = 8
V = 512
L = 16
NC, NS = 2, 16
NW = NC * NS
ROWS_PER_W = (B * N1) // NW
TILES_PER_B = N1 // ROWS_PER_W
G = 32
NGROUPS = ROWS_PER_W // G
GROUP_WORDS = G * N1
STAGE_LEN = ROWS_PER_W * N + 16


def _tile_body(pl_hbm, emb_hbm, out_hbm, idx_buf, out_bufs, emb_v, sem0, sem1):
    wid = lax.axis_index("s") * NC + lax.axis_index("c")
    b = wid // TILES_PER_B
    i_start = (wid % TILES_PER_B) * ROWS_PER_W
    start_p = jnp.maximum(i_start - 1, 0)
    offset = (b * N + start_p) * N
    start8 = (offset // 8) * 8
    slop = offset - start8
    arow = (i_start - 1 - start_p) * N + slop - i_start * N

    pltpu.sync_copy(emb_hbm, emb_v)
    pltpu.sync_copy(pl_hbm.at[pl.ds(start8, STAGE_LEN)], idx_buf)

    iota = lax.iota(jnp.int32, L)
    ones = jnp.full((L,), 1, jnp.int32)
    emb1 = plsc.load_gather(emb_v, [ones])

    @pl.when(i_start == 0)
    def _():
        for j in range(N1 // L):
            out_bufs[0, pl.ds(j * L, L)] = emb1

    descs = [None] * NGROUPS
    for g in range(NGROUPS):
        buf = g & 1
        if g >= 2:
            for d in descs[g - 2]:
                d.wait()
        lo = jnp.where(i_start == 0, 1, 0) if g == 0 else 0

        @pl.loop(lo, G)
        def _row(lr):
            i = i_start + g * G + lr
            base = arow + i * N
            obase = lr * N1
            idx0 = jnp.maximum(iota + (base - 1), 0)
            v0 = plsc.load_gather(idx_buf, [idx0])
            v0 = jnp.where(iota == 0, 1, v0)
            out_bufs[buf, pl.ds(obase, L)] = plsc.load_gather(emb_v, [v0])
            for j in range(1, N1 // L):
                idxs = iota + (base + j * L - 1)
                vals = plsc.load_gather(idx_buf, [idxs])
                out_bufs[buf, pl.ds(obase + j * L, L)] = plsc.load_gather(emb_v, [vals])

        row0 = i_start + g * G
        descs[g] = [
            pltpu.async_copy(
                out_bufs.at[buf],
                out_hbm.at[pl.ds(((b * H + h) * N1 + row0) * N1, GROUP_WORDS)],
                sem0 if buf == 0 else sem1,
            )
            for h in range(H)
        ]
    for g in (NGROUPS - 2, NGROUPS - 1):
        for d in descs[g]:
            d.wait()


def kernel(X, mask, path_lengths, path_length_embedding):
    del X, mask
    pl_flat = path_lengths.reshape(-1)
    emb_flat = path_length_embedding.reshape(-1)

    mesh = plsc.VectorSubcoreMesh(
        core_axis_name="c", subcore_axis_name="s", num_cores=NC, num_subcores=NS
    )
    run = functools.partial(
        pl.kernel,
        out_type=jax.ShapeDtypeStruct((B * H * N1 * N1,), jnp.float32),
        mesh=mesh,
        scratch_types=[
            pltpu.VMEM((STAGE_LEN,), jnp.int32),
            pltpu.VMEM((2, GROUP_WORDS), jnp.float32),
            pltpu.VMEM((V,), jnp.float32),
            pltpu.SemaphoreType.DMA,
            pltpu.SemaphoreType.DMA,
        ],
        compiler_params=pltpu.CompilerParams(needs_layout_passes=False),
    )(_tile_body)
    out = run(pl_flat, emb_flat)
    return out.reshape(B * H, N1, N1)


# speedup vs baseline: 247.5489x; 1.9428x over previous
import functools

import jax
import jax.numpy as jnp
from jax import lax
from jax.experimental import pallas as pl
from jax.experimental.pallas import tpu as pltpu
from jax.experimental.pallas import tpu_sc as plsc

B = 8
N = 511
N1 = 512
H = 8
V = 512
L = 16
NC, NS = 2, 16
NW = NC * NS
ROWS_PER_W = (B * N1) // NW
TILES_PER_B = N1 // ROWS_PER_W
G = 32
NGROUPS = ROWS_PER_W // G
GROUP_WORDS = G * N1
STAGE_LEN = ROWS_PER_W * N + 16


def _tile_body(pl_hbm, emb_hbm, out_hbm, idx_buf, out_bufs, emb_v, sem0, sem1):
    wid = lax.axis_index("s") * NC + lax.axis_index("c")
    b = wid // TILES_PER_B
    i_start = (wid % TILES_PER_B) * ROWS_PER_W
    start_p = jnp.maximum(i_start - 1, 0)
    offset = (b * N + start_p) * N
    start8 = (offset // 8) * 8
    slop = offset - start8
    arow = (i_start - 1 - start_p) * N + slop - i_start * N

    pltpu.sync_copy(emb_hbm, emb_v)
    pltpu.sync_copy(pl_hbm.at[pl.ds(start8, STAGE_LEN)], idx_buf)

    iota = lax.iota(jnp.int32, L)
    ones = jnp.full((L,), 1, jnp.int32)
    emb1 = plsc.load_gather(emb_v, [ones])

    @pl.when(i_start == 0)
    def _():
        for j in range(N1 // L):
            out_bufs[0, 0, pl.ds(j * L, L)] = emb1

    descs = [None] * NGROUPS
    for g in range(NGROUPS):
        buf = g & 1
        if g >= 2:
            for d in descs[g - 2]:
                d.wait()
        lo = jnp.where(i_start == 0, 1, 0) if g == 0 else 0

        @pl.loop(lo, G)
        def _row(lr):
            i = i_start + g * G + lr
            base = arow + i * N
            idx0 = jnp.maximum(iota + (base - 1), 0)
            v0 = plsc.load_gather(idx_buf, [idx0])
            v0 = jnp.where(iota == 0, 1, v0)
            out_bufs[buf, lr, pl.ds(0, L)] = plsc.load_gather(emb_v, [v0])
            for j in range(1, N1 // L):
                idxs = iota + (base + j * L - 1)
                vals = plsc.load_gather(idx_buf, [idxs])
                out_bufs[buf, lr, pl.ds(j * L, L)] = plsc.load_gather(emb_v, [vals])

        row0 = i_start + g * G
        descs[g] = [
            pltpu.async_copy(
                out_bufs.at[buf],
                out_hbm.at[b * H + h, pl.ds(row0, G), :],
                sem0 if buf == 0 else sem1,
            )
            for h in range(H)
        ]
    for g in (NGROUPS - 2, NGROUPS - 1):
        for d in descs[g]:
            d.wait()


def kernel(X, mask, path_lengths, path_length_embedding):
    del X, mask
    pl_flat = path_lengths.reshape(-1)
    emb_flat = path_length_embedding.reshape(-1)

    mesh = plsc.VectorSubcoreMesh(
        core_axis_name="c", subcore_axis_name="s", num_cores=NC, num_subcores=NS
    )
    run = functools.partial(
        pl.kernel,
        out_type=jax.ShapeDtypeStruct((B * H, N1, N1), jnp.float32),
        mesh=mesh,
        scratch_types=[
            pltpu.VMEM((STAGE_LEN,), jnp.int32),
            pltpu.VMEM((2, G, N1), jnp.float32),
            pltpu.VMEM((V,), jnp.float32),
            pltpu.SemaphoreType.DMA,
            pltpu.SemaphoreType.DMA,
        ],
        compiler_params=pltpu.CompilerParams(needs_layout_passes=False),
    )(_tile_body)
    return run(pl_flat, emb_flat)


# speedup vs baseline: 287.4911x; 1.1614x over previous
import functools

import jax
import jax.numpy as jnp
from jax import lax
from jax.experimental import pallas as pl
from jax.experimental.pallas import tpu as pltpu
from jax.experimental.pallas import tpu_sc as plsc

B = 8
N = 511
N1 = 512
H = 8
V = 512
L = 16
NC, NS = 2, 16
NW = NC * NS
ROWS_PER_W = (B * N1) // NW
TILES_PER_B = N1 // ROWS_PER_W
G = 32
NGROUPS = ROWS_PER_W // G
GROUP_WORDS = G * N1


def _gather_row(src_buf, rowv, emb_v, iota, out_bufs, buf, lr):
    col0 = jnp.maximum(iota - 1, 0)
    v0 = plsc.load_gather(src_buf, [rowv, col0])
    v0 = jnp.where(iota == 0, 1, v0)
    out_bufs[buf, lr, pl.ds(0, L)] = plsc.load_gather(emb_v, [v0])
    for j in range(1, N1 // L):
        colj = iota + (j * L - 1)
        vals = plsc.load_gather(src_buf, [rowv, colj])
        out_bufs[buf, lr, pl.ds(j * L, L)] = plsc.load_gather(emb_v, [vals])


def _tile_body(pl_hbm, emb_hbm, out_hbm, idx_buf, bnd_buf, out_bufs, emb_v,
               sem0, sem1):
    wid = lax.axis_index("s") * NC + lax.axis_index("c")
    b = wid // TILES_PER_B
    i_start = (wid % TILES_PER_B) * ROWS_PER_W

    pltpu.sync_copy(emb_hbm, emb_v)
    pltpu.sync_copy(
        pl_hbm.at[b, pl.ds(pl.multiple_of(i_start, 8), ROWS_PER_W), :], idx_buf
    )
    bnd_start = pl.multiple_of(jnp.maximum(i_start - 8, 0), 8)
    pltpu.sync_copy(pl_hbm.at[b, pl.ds(bnd_start, 8), :], bnd_buf)

    iota = lax.iota(jnp.int32, L)
    ones = jnp.full((L,), 1, jnp.int32)
    emb1 = plsc.load_gather(emb_v, [ones])

    @pl.when(i_start == 0)
    def _():
        for j in range(N1 // L):
            out_bufs[0, 0, pl.ds(j * L, L)] = emb1

    @pl.when(i_start > 0)
    def _():
        _gather_row(bnd_buf, jnp.full((L,), 7, jnp.int32), emb_v, iota,
                    out_bufs, 0, 0)

    descs = [None] * NGROUPS
    for g in range(NGROUPS):
        buf = g & 1
        if g >= 2:
            for d in descs[g - 2]:
                d.wait()
        lo = 1 if g == 0 else 0

        @pl.loop(lo, G)
        def _row(lr):
            lrow = g * G + lr - 1
            _gather_row(idx_buf, jnp.full((L,), lrow, jnp.int32), emb_v, iota,
                        out_bufs, buf, lr)

        row0 = i_start + g * G
        descs[g] = [
            pltpu.async_copy(
                out_bufs.at[buf],
                out_hbm.at[b * H + h, pl.ds(row0, G), :],
                sem0 if buf == 0 else sem1,
            )
            for h in range(H)
        ]
    for g in (NGROUPS - 2, NGROUPS - 1):
        for d in descs[g]:
            d.wait()


def kernel(X, mask, path_lengths, path_length_embedding):
    del X, mask
    emb_flat = path_length_embedding.reshape(-1)

    mesh = plsc.VectorSubcoreMesh(
        core_axis_name="c", subcore_axis_name="s", num_cores=NC, num_subcores=NS
    )
    run = functools.partial(
        pl.kernel,
        out_type=jax.ShapeDtypeStruct((B * H, N1, N1), jnp.float32),
        mesh=mesh,
        scratch_types=[
            pltpu.VMEM((ROWS_PER_W, N), jnp.int32),
            pltpu.VMEM((8, N), jnp.int32),
            pltpu.VMEM((2, G, N1), jnp.float32),
            pltpu.VMEM((V,), jnp.float32),
            pltpu.SemaphoreType.DMA,
            pltpu.SemaphoreType.DMA,
        ],
        compiler_params=pltpu.CompilerParams(needs_layout_passes=False),
    )(_tile_body)
    return run(path_lengths, emb_flat)


# speedup vs baseline: 324.8825x; 1.1301x over previous
import functools

import jax
import jax.numpy as jnp
from jax import lax
from jax.experimental import pallas as pl
from jax.experimental.pallas import tpu as pltpu
from jax.experimental.pallas import tpu_sc as plsc

B = 8
N = 511
N1 = 512
H = 8
V = 512
L = 16
NC, NS = 2, 16
NW = NC * NS
ROWS_PER_W = (B * N1) // NW
TILES_PER_B = N1 // ROWS_PER_W
G = 32
NGROUPS = ROWS_PER_W // G
GROUP_WORDS = G * N1


def _gather_row(src_buf, rowv, emb_v, iota, out_bufs, buf, lr):
    col0 = jnp.maximum(iota - 1, 0)
    v0 = plsc.load_gather(src_buf, [rowv, col0])
    v0 = jnp.where(iota == 0, 1, v0)
    out_bufs[buf, lr, pl.ds(0, L)] = plsc.load_gather(emb_v, [v0])
    for j in range(1, N1 // L):
        colj = iota + (j * L - 1)
        vals = plsc.load_gather(src_buf, [rowv, colj])
        out_bufs[buf, lr, pl.ds(j * L, L)] = plsc.load_gather(emb_v, [vals])


def _tile_body(pl_hbm, emb_hbm, out_hbm, idx_buf, out_bufs, emb_v, sem0, sem1):
    wid = lax.axis_index("s") * NC + lax.axis_index("c")
    b = wid // TILES_PER_B
    i_start = (wid % TILES_PER_B) * ROWS_PER_W
    start_p = jnp.maximum(i_start - 1, 0)

    pltpu.sync_copy(emb_hbm, emb_v)
    pltpu.sync_copy(pl_hbm.at[pl.ds(start_p, ROWS_PER_W), b, :], idx_buf)

    iota = lax.iota(jnp.int32, L)
    ones = jnp.full((L,), 1, jnp.int32)
    emb1 = plsc.load_gather(emb_v, [ones])

    @pl.when(i_start == 0)
    def _():
        for j in range(N1 // L):
            out_bufs[0, 0, pl.ds(j * L, L)] = emb1

    descs = [None] * NGROUPS
    for g in range(NGROUPS):
        buf = g & 1
        if g >= 2:
            for d in descs[g - 2]:
                d.wait()
        lo = jnp.where(i_start == 0, 1, 0) if g == 0 else 0

        @pl.loop(lo, G)
        def _row(lr):
            i = i_start + g * G + lr
            rowv = jnp.full((L,), i - 1 - start_p, jnp.int32)
            _gather_row(idx_buf, rowv, emb_v, iota, out_bufs, buf, lr)

        row0 = i_start + g * G
        descs[g] = [
            pltpu.async_copy(
                out_bufs.at[buf],
                out_hbm.at[b * H + h, pl.ds(row0, G), :],
                sem0 if buf == 0 else sem1,
            )
            for h in range(H)
        ]
    for g in (NGROUPS - 2, NGROUPS - 1):
        for d in descs[g]:
            d.wait()


def kernel(X, mask, path_lengths, path_length_embedding):
    del X, mask
    emb_flat = path_length_embedding.reshape(-1)

    mesh = plsc.VectorSubcoreMesh(
        core_axis_name="c", subcore_axis_name="s", num_cores=NC, num_subcores=NS
    )
    run = functools.partial(
        pl.kernel,
        out_type=jax.ShapeDtypeStruct((B * H, N1, N1), jnp.float32),
        mesh=mesh,
        scratch_types=[
            pltpu.VMEM((ROWS_PER_W, N), jnp.int32),
            pltpu.VMEM((2, G, N1), jnp.float32),
            pltpu.VMEM((V,), jnp.float32),
            pltpu.SemaphoreType.DMA,
            pltpu.SemaphoreType.DMA,
        ],
        compiler_params=pltpu.CompilerParams(needs_layout_passes=False),
    )(_tile_body)
    return run(jnp.transpose(path_lengths, (1, 0, 2)), emb_flat)


# speedup vs baseline: 335.4678x; 1.0326x over previous
import functools

import jax
import jax.numpy as jnp
from jax import lax
from jax.experimental import pallas as pl
from jax.experimental.pallas import tpu as pltpu
from jax.experimental.pallas import tpu_sc as plsc

B = 8
N = 511
N1 = 512
H = 8
V = 512
L = 16
NC, NS = 2, 16
NW = NC * NS
ROWS_PER_W = (B * N1) // NW
TILES_PER_B = N1 // ROWS_PER_W
G = 16
NGROUPS = ROWS_PER_W // G
GROUP_WORDS = G * N1


def _gather_row(src_buf, rowv, emb_v, iota, out_bufs, buf, lr):
    col0 = jnp.maximum(iota - 1, 0)
    v0 = plsc.load_gather(src_buf, [rowv, col0])
    v0 = jnp.where(iota == 0, 1, v0)
    out_bufs[buf, lr, pl.ds(0, L)] = plsc.load_gather(emb_v, [v0])
    for j in range(1, N1 // L):
        colj = iota + (j * L - 1)
        vals = plsc.load_gather(src_buf, [rowv, colj])
        out_bufs[buf, lr, pl.ds(j * L, L)] = plsc.load_gather(emb_v, [vals])


def _tile_body(pl_hbm, emb_hbm, out_hbm, idx_buf, out_bufs, emb_v,
               sem0, sem1, sem_stage):
    wid = lax.axis_index("s") * NC + lax.axis_index("c")
    b = wid // TILES_PER_B
    i_start = (wid % TILES_PER_B) * ROWS_PER_W
    start_p = jnp.maximum(i_start - 1, 0)

    pltpu.sync_copy(emb_hbm, emb_v)
    stage_head = 24
    pltpu.sync_copy(pl_hbm.at[pl.ds(start_p, stage_head), b, :],
                    idx_buf.at[pl.ds(0, stage_head), :])
    stage_rest = pltpu.async_copy(
        pl_hbm.at[pl.ds(start_p + stage_head, ROWS_PER_W - stage_head), b, :],
        idx_buf.at[pl.ds(stage_head, ROWS_PER_W - stage_head), :],
        sem_stage,
    )

    iota = lax.iota(jnp.int32, L)
    ones = jnp.full((L,), 1, jnp.int32)
    emb1 = plsc.load_gather(emb_v, [ones])

    @pl.when(i_start == 0)
    def _():
        for j in range(N1 // L):
            out_bufs[0, 0, pl.ds(j * L, L)] = emb1

    descs = [None] * NGROUPS
    for g in range(NGROUPS):
        buf = g & 1
        if g == 1:
            stage_rest.wait()
        if g >= 2:
            for d in descs[g - 2]:
                d.wait()
        lo = jnp.where(i_start == 0, 1, 0) if g == 0 else 0

        @pl.loop(lo, G)
        def _row(lr):
            i = i_start + g * G + lr
            rowv = jnp.full((L,), i - 1 - start_p, jnp.int32)
            _gather_row(idx_buf, rowv, emb_v, iota, out_bufs, buf, lr)

        row0 = i_start + g * G
        descs[g] = [
            pltpu.async_copy(
                out_bufs.at[buf],
                out_hbm.at[b * H + h, pl.ds(row0, G), :],
                sem0 if buf == 0 else sem1,
            )
            for h in range(H)
        ]
    for g in (NGROUPS - 2, NGROUPS - 1):
        for d in descs[g]:
            d.wait()


def kernel(X, mask, path_lengths, path_length_embedding):
    del X, mask
    emb_flat = path_length_embedding.reshape(-1)

    mesh = plsc.VectorSubcoreMesh(
        core_axis_name="c", subcore_axis_name="s", num_cores=NC, num_subcores=NS
    )
    run = functools.partial(
        pl.kernel,
        out_type=jax.ShapeDtypeStruct((B * H, N1, N1), jnp.float32),
        mesh=mesh,
        scratch_types=[
            pltpu.VMEM((ROWS_PER_W, N), jnp.int32),
            pltpu.VMEM((2, G, N1), jnp.float32),
            pltpu.VMEM((V,), jnp.float32),
            pltpu.SemaphoreType.DMA,
            pltpu.SemaphoreType.DMA,
            pltpu.SemaphoreType.DMA,
        ],
        compiler_params=pltpu.CompilerParams(needs_layout_passes=False),
    )(_tile_body)
    return run(jnp.transpose(path_lengths, (1, 0, 2)), emb_flat)
